# baseline (speedup 1.0000x reference)
import functools

import jax
import jax.numpy as jnp
from jax.experimental import pallas as pl
from jax.experimental.pallas import tpu as pltpu

E = 8
TOP = 2
C = 192
G = 4
B = 8
H = 16
W = 16
CCAP = 192
HW = H * W
BHW = B * HW


def _shift_hw(x4, sh, sw):
    if sh > 0:
        x4 = jnp.concatenate([x4[:, sh:], jnp.zeros_like(x4[:, :sh])], axis=1)
    elif sh < 0:
        x4 = jnp.concatenate([jnp.zeros_like(x4[:, sh:]), x4[:, :sh]], axis=1)
    if sw > 0:
        x4 = jnp.concatenate([x4[:, :, sw:], jnp.zeros_like(x4[:, :, :sw])], axis=2)
    elif sw < 0:
        x4 = jnp.concatenate([jnp.zeros_like(x4[:, :, sw:]), x4[:, :, :sw]], axis=2)
    return x4


def _routing_body(x_ref, gates_ref, cw_ref, mask_ref, loss_ref):
    x_gap = jnp.mean(x_ref[...], axis=1)
    eio = jax.lax.broadcasted_iota(jnp.int32, (B, E), 1)
    loss_acc = jnp.float32(0.0)
    nsel = jnp.zeros((B, E), jnp.int32)
    for g in range(G):
        logits = jnp.dot(x_gap, gates_ref[g], preferred_element_type=jnp.float32)
        m = jnp.max(logits, axis=1, keepdims=True)
        ex = jnp.exp(logits - m)
        probs = ex / jnp.sum(ex, axis=1, keepdims=True)
        usage = jnp.sum(probs, axis=0)
        mu = jnp.mean(usage)
        var = jnp.mean((usage - mu) ** 2)
        loss_acc = loss_acc + var / (mu * mu + 1e-10)
        v1 = jnp.max(probs, axis=1, keepdims=True)
        i1 = jnp.min(jnp.where(probs == v1, eio, E + 1), axis=1, keepdims=True)
        p2 = jnp.where(eio == i1, -1.0, probs)
        v2 = jnp.max(p2, axis=1, keepdims=True)
        i2 = jnp.min(jnp.where(p2 == v2, eio, E + 1), axis=1, keepdims=True)
        t = jnp.exp(v2 - v1)
        w1 = 1.0 / (1.0 + t)
        w2 = t / (1.0 + t)
        sel = (eio == i1) | (eio == i2)
        nsel = nsel + sel.astype(jnp.int32)
        cw = jnp.where(eio == i1, w1, jnp.float32(0.0)) \
            + jnp.where(eio == i2, w2, jnp.float32(0.0))
        cw_ref[:, g:g + 1, :] = jnp.transpose(cw).reshape(E, 1, B)
    mask_ref[...] = jnp.transpose((nsel > 0).astype(jnp.int32))
    loss_ref[...] = jnp.broadcast_to(loss_acc / G, (1, 1))


def _expert_body(mask_sref, xb_ref, wc_ref, bc_ref, wp_ref, bp_ref, cw_ref,
                 ys_ref):
    e = pl.program_id(0)
    b = pl.program_id(1)

    @pl.when((e == 0) & (b == 0))
    def _init():
        ys_ref[...] = jnp.zeros((G, BHW, C), jnp.float32)

    @pl.when(mask_sref[e, b] != 0)
    def _compute():
        x4 = xb_ref[...].reshape(1, H, W, C)
        acc = jnp.zeros((HW, CCAP), jnp.float32)
        for dy in range(3):
            for dx in range(3):
                xs = _shift_hw(x4, dy - 1, dx - 1).reshape(HW, C)
                acc = acc + jnp.dot(xs, wc_ref[0, dy, dx],
                                    preferred_element_type=jnp.float32)
        u = acc + bc_ref[0]
        sn = jnp.sum(u * u, axis=1, keepdims=True)
        scale = sn / ((1.0 + sn) * (jnp.sqrt(sn) + 1e-8))
        u = (scale * u).astype(jnp.bfloat16)
        out2d = jnp.dot(u, wp_ref[0], preferred_element_type=jnp.float32) \
            + bp_ref[0]
        bio = jax.lax.broadcasted_iota(jnp.int32, (1, B), 1)
        for g in range(G):
            wvec = cw_ref[0, g].reshape(1, B)
            w = jnp.sum(jnp.where(bio == b, wvec, 0.0))
            rows = ys_ref[g, pl.ds(b * HW, HW), :]
            ys_ref[g, pl.ds(b * HW, HW), :] = rows + w * out2d


@jax.jit
def _moe(x, Wc, bc, Wp, bp, gates):
    x3 = jnp.transpose(x, (0, 2, 3, 1)).reshape(B, HW, C)
    x3b = x3.astype(jnp.bfloat16)
    Wc_r = jnp.transpose(Wc.astype(jnp.bfloat16), (0, 3, 4, 2, 1))
    bc_r = bc.reshape(E, 1, CCAP)
    Wp_r = jnp.transpose(Wp[..., 0, 0].astype(jnp.bfloat16), (0, 2, 1))
    bp_r = bp.reshape(E, 1, C)

    cw, mask, loss = pl.pallas_call(
        _routing_body,
        grid=(1,),
        in_specs=[
            pl.BlockSpec((B, HW, C), lambda i: (0, 0, 0)),
            pl.BlockSpec((G, C, E), lambda i: (0, 0, 0)),
        ],
        out_specs=[
            pl.BlockSpec((E, G, B), lambda i: (0, 0, 0)),
            pl.BlockSpec((E, B), lambda i: (0, 0)),
            pl.BlockSpec((1, 1), lambda i: (0, 0)),
        ],
        out_shape=[
            jax.ShapeDtypeStruct((E, G, B), jnp.float32),
            jax.ShapeDtypeStruct((E, B), jnp.int32),
            jax.ShapeDtypeStruct((1, 1), jnp.float32),
        ],
    )(x3, gates)

    ys = pl.pallas_call(
        _expert_body,
        grid_spec=pltpu.PrefetchScalarGridSpec(
            num_scalar_prefetch=1,
            grid=(E, B),
            in_specs=[
                pl.BlockSpec((1, HW, C), lambda e, b, m: (b, 0, 0)),
                pl.BlockSpec((1, 3, 3, C, CCAP), lambda e, b, m: (e, 0, 0, 0, 0)),
                pl.BlockSpec((1, 1, CCAP), lambda e, b, m: (e, 0, 0)),
                pl.BlockSpec((1, CCAP, C), lambda e, b, m: (e, 0, 0)),
                pl.BlockSpec((1, 1, C), lambda e, b, m: (e, 0, 0)),
                pl.BlockSpec((1, G, B), lambda e, b, m: (e, 0, 0)),
            ],
            out_specs=pl.BlockSpec((G, BHW, C), lambda e, b, m: (0, 0, 0)),
        ),
        out_shape=jax.ShapeDtypeStruct((G, BHW, C), jnp.float32),
        compiler_params=pltpu.CompilerParams(
            dimension_semantics=("arbitrary", "arbitrary"),
        ),
    )(mask, x3b, Wc_r, bc_r, Wp_r, bp_r, cw)

    ys4 = jnp.transpose(ys.reshape(G, B, H, W, C), (0, 1, 4, 2, 3))
    return ys4[0], ys4[1], ys4[2], ys4[3], loss[0, 0]


def kernel(x, Wc, bc, Wp, bp, gates):
    return _moe(x, Wc, bc, Wp, bp, gates)


# speedup vs baseline: 1.2332x; 1.2332x over previous
import functools

import jax
import jax.numpy as jnp
from jax.experimental import pallas as pl
from jax.experimental.pallas import tpu as pltpu

E = 8
TOP = 2
C = 192
G = 4
B = 8
H = 16
W = 16
CCAP = 192
HW = H * W
BHW = B * HW


def _moe_body(x_ref, xb_ref, gates_ref, wc_ref, bc_ref, wp_ref, bp_ref,
              ys_ref, loss_ref, xs_ref):
    rio = jax.lax.broadcasted_iota(jnp.int32, (1, BHW), 1)
    pool = (jax.lax.broadcasted_iota(jnp.int32, (BHW, B), 0) // HW ==
            jax.lax.broadcasted_iota(jnp.int32, (BHW, B), 1))
    poolf = pool.astype(jnp.float32)
    x_gap = jnp.dot(x_ref[...], poolf,
                    preferred_element_type=jnp.float32) * (1.0 / HW)
    eio = jax.lax.broadcasted_iota(jnp.int32, (E, B), 0)
    loss_acc = jnp.float32(0.0)
    wrows = []
    for g in range(G):
        logits = jnp.dot(gates_ref[g], x_gap,
                         preferred_element_type=jnp.float32)
        m = jnp.max(logits, axis=0, keepdims=True)
        ex = jnp.exp(logits - m)
        probs = ex / jnp.sum(ex, axis=0, keepdims=True)
        usage = jnp.sum(probs, axis=1)
        mu = jnp.mean(usage)
        var = jnp.mean((usage - mu) ** 2)
        loss_acc = loss_acc + var / (mu * mu + 1e-10)
        v1 = jnp.max(probs, axis=0, keepdims=True)
        i1 = jnp.min(jnp.where(probs == v1, eio, E + 1), axis=0, keepdims=True)
        p2 = jnp.where(eio == i1, -1.0, probs)
        v2 = jnp.max(p2, axis=0, keepdims=True)
        i2 = jnp.min(jnp.where(p2 == v2, eio, E + 1), axis=0, keepdims=True)
        t = jnp.exp(v2 - v1)
        w1 = 1.0 / (1.0 + t)
        w2 = t / (1.0 + t)
        cw = jnp.where(eio == i1, w1, jnp.float32(0.0)) \
            + jnp.where(eio == i2, w2, jnp.float32(0.0))
        wrows.append(jnp.dot(cw, jnp.transpose(poolf),
                             preferred_element_type=jnp.float32))
    loss_ref[...] = jnp.broadcast_to(loss_acc / G, (1, 1))

    xb = xb_ref[...]
    hpos = (rio // W) % H
    wpos = rio % W
    for dy in range(3):
        for dx in range(3):
            k = dy * 3 + dx
            sh, sw = dy - 1, dx - 1
            shift = sh * W + sw
            rolled = jnp.roll(xb, -shift, axis=1) if shift != 0 else xb
            mask = jnp.ones((1, BHW), jnp.bool_)
            if sh > 0:
                mask = mask & (hpos < H - sh)
            elif sh < 0:
                mask = mask & (hpos >= -sh)
            if sw > 0:
                mask = mask & (wpos < W - sw)
            elif sw < 0:
                mask = mask & (wpos >= -sw)
            xs_ref[k * C:(k + 1) * C, :] = rolled * mask.astype(jnp.bfloat16)

    xs = xs_ref[...]
    for e in range(E):
        u = jnp.dot(wc_ref[e], xs, preferred_element_type=jnp.float32)
        u = u + bc_ref[e]
        sn = jnp.sum(u * u, axis=0, keepdims=True)
        scale = sn / ((1.0 + sn) * (jnp.sqrt(sn) + 1e-8))
        u = (scale * u).astype(jnp.bfloat16)
        out = jnp.dot(wp_ref[e], u, preferred_element_type=jnp.float32) \
            + bp_ref[e]
        for g in range(G):
            contrib = wrows[g][e:e + 1, :] * out
            if e == 0:
                ys_ref[g] = contrib
            else:
                ys_ref[g] = ys_ref[g] + contrib


@jax.jit
def _moe(x, Wc, bc, Wp, bp, gates):
    xT = jnp.transpose(x.reshape(B, C, HW), (1, 0, 2)).reshape(C, BHW)
    xTb = xT.astype(jnp.bfloat16)
    Wc_r = jnp.transpose(Wc.astype(jnp.bfloat16),
                         (0, 1, 3, 4, 2)).reshape(E, CCAP, 9 * C)
    bc_r = bc.reshape(E, CCAP, 1)
    Wp_r = Wp[..., 0, 0].astype(jnp.bfloat16)
    bp_r = bp.reshape(E, C, 1)
    gates_r = jnp.transpose(gates, (0, 2, 1))

    ys, loss = pl.pallas_call(
        _moe_body,
        grid=(1,),
        in_specs=[
            pl.BlockSpec((C, BHW), lambda i: (0, 0)),
            pl.BlockSpec((C, BHW), lambda i: (0, 0)),
            pl.BlockSpec((G, E, C), lambda i: (0, 0, 0)),
            pl.BlockSpec((E, CCAP, 9 * C), lambda i: (0, 0, 0)),
            pl.BlockSpec((E, CCAP, 1), lambda i: (0, 0, 0)),
            pl.BlockSpec((E, C, CCAP), lambda i: (0, 0, 0)),
            pl.BlockSpec((E, C, 1), lambda i: (0, 0, 0)),
        ],
        out_specs=[
            pl.BlockSpec((G, C, BHW), lambda i: (0, 0, 0)),
            pl.BlockSpec((1, 1), lambda i: (0, 0)),
        ],
        out_shape=[
            jax.ShapeDtypeStruct((G, C, BHW), jnp.float32),
            jax.ShapeDtypeStruct((1, 1), jnp.float32),
        ],
        scratch_shapes=[pltpu.VMEM((9 * C, BHW), jnp.bfloat16)],
        compiler_params=pltpu.CompilerParams(
            dimension_semantics=("arbitrary",),
        ),
    )(xT, xTb, gates_r, Wc_r, bc_r, Wp_r, bp_r)

    ys4 = jnp.transpose(ys.reshape(G, C, B, H, W), (0, 2, 1, 3, 4))
    return ys4[0], ys4[1], ys4[2], ys4[3], loss[0, 0]


def kernel(x, Wc, bc, Wp, bp, gates):
    return _moe(x, Wc, bc, Wp, bp, gates)
